# Initial kernel scaffold; baseline (speedup 1.0000x reference)
import functools

import jax
import jax.numpy as jnp
from jax import lax
from jax.experimental import pallas as pl
from jax.experimental.pallas import tpu as pltpu
from jax.experimental.pallas import tpu_sc as plsc

N = 10000
E = 320000
LANES = 16
EDGE_COLS = 128
TILES = 16
ROWS_PER_TILE = 160
ROWS_TOTAL = TILES * ROWS_PER_TILE
E_PAD = ROWS_TOTAL * EDGE_COLS
NODES_PER_TILE = N // TILES
ZCHUNK = 125

BN = 1000


@functools.cache
def _sc_agg(dh):
    mesh = plsc.VectorSubcoreMesh(core_axis_name="c", subcore_axis_name="s")

    @functools.partial(
        pl.kernel,
        mesh=mesh,
        out_type=jax.ShapeDtypeStruct((2 * N, dh), jnp.float32),
        scratch_types=[
            pltpu.VMEM((ROWS_PER_TILE, EDGE_COLS), jnp.int32),
            pltpu.VMEM((ROWS_PER_TILE, EDGE_COLS), jnp.int32),
            pltpu.VMEM((ROWS_PER_TILE, EDGE_COLS), jnp.float32),
            pltpu.VMEM((EDGE_COLS, dh), jnp.float32),
            pltpu.VMEM((ZCHUNK, dh), jnp.float32),
            pltpu.VMEM_SHARED((N, dh), jnp.float32),
            pltpu.SemaphoreType.DMA,
        ],
    )
    def agg(ys_hbm, src_hbm, dst_hbm, ew_hbm, out_hbm,
            src_v, dst_v, ew_v, rows_v, zero_v, acc_sh, sem):
        c = lax.axis_index("c")
        s = lax.axis_index("s")

        zf = jnp.zeros((LANES,), jnp.float32)

        def zrow(r, carry):
            for k in range(dh // LANES):
                zero_v[r, pl.ds(k * LANES, LANES)] = zf
            return carry

        lax.fori_loop(0, ZCHUNK, zrow, 0)
        n0 = s * NODES_PER_TILE
        for t in range(NODES_PER_TILE // ZCHUNK):
            pltpu.sync_copy(zero_v, acc_sh.at[pl.ds(n0 + t * ZCHUNK, ZCHUNK)])
        plsc.subcore_barrier()

        row0 = s * ROWS_PER_TILE
        pltpu.sync_copy(src_hbm.at[c, pl.ds(row0, ROWS_PER_TILE)], src_v)
        pltpu.sync_copy(dst_hbm.at[pl.ds(row0, ROWS_PER_TILE)], dst_v)
        pltpu.sync_copy(ew_hbm.at[pl.ds(row0, ROWS_PER_TILE)], ew_v)

        def body(j, carry):
            pltpu.async_copy(ys_hbm.at[src_v.at[j]], rows_v, sem).wait()

            def scale(e, inner):
                w = ew_v[j, e]
                for k in range(dh // LANES):
                    rows_v[e, pl.ds(k * LANES, LANES)] = (
                        rows_v[e, pl.ds(k * LANES, LANES)] * w)
                return inner

            lax.fori_loop(0, EDGE_COLS, scale, 0)

            pltpu.sync_copy(rows_v, acc_sh.at[dst_v.at[j]], add=True)
            return carry

        lax.fori_loop(0, ROWS_PER_TILE, body, 0)
        plsc.subcore_barrier()

        pltpu.sync_copy(acc_sh.at[pl.ds(n0, NODES_PER_TILE)],
                        out_hbm.at[pl.ds(c * N + n0, NODES_PER_TILE)])

    return agg


def _tc_first(x, w1, deg16):
    def body(x_ref, w_ref, deg_ref, ys_ref, dis_ref):
        dis = lax.rsqrt(deg_ref[:, 0:1] + 1.0)
        xw = jnp.dot(x_ref[...], w_ref[...],
                     preferred_element_type=jnp.float32)
        ys = xw * dis
        ys_ref[0] = ys[:, :128]
        ys_ref[1] = ys[:, 128:]
        dis_ref[...] = dis

    return pl.pallas_call(
        body,
        grid=(N // BN,),
        in_specs=[
            pl.BlockSpec((BN, 128), lambda i: (i, 0)),
            pl.BlockSpec((128, 256), lambda i: (0, 0)),
            pl.BlockSpec((BN, 16), lambda i: (i, 0)),
        ],
        out_specs=[
            pl.BlockSpec((2, BN, 128), lambda i: (0, i, 0)),
            pl.BlockSpec((BN, 1), lambda i: (i, 0)),
        ],
        out_shape=[
            jax.ShapeDtypeStruct((2, N, 128), jnp.float32),
            jax.ShapeDtypeStruct((N, 1), jnp.float32),
        ],
    )(x, w1, deg16)


def _tc_mid(agg, ys, dis2, b2d, w, d_in_h, d_out):
    doh = d_out // 2

    def body(a_ref, ys_ref, dis_ref, b_ref, w_ref, out_ref):
        dis = dis_ref[...]
        h0 = jnp.maximum((a_ref[0] + ys_ref[0]) * dis + b_ref[0], 0.0)
        h1 = jnp.maximum((a_ref[1] + ys_ref[1]) * dis + b_ref[1], 0.0)
        out = jnp.dot(h0, w_ref[:d_in_h, :],
                      preferred_element_type=jnp.float32)
        out = out + jnp.dot(h1, w_ref[d_in_h:, :],
                            preferred_element_type=jnp.float32)
        ysn = out * dis
        out_ref[0] = ysn[:, :doh]
        out_ref[1] = ysn[:, doh:]

    return pl.pallas_call(
        body,
        grid=(N // BN,),
        in_specs=[
            pl.BlockSpec((2, BN, d_in_h), lambda i: (0, i, 0)),
            pl.BlockSpec((2, BN, d_in_h), lambda i: (0, i, 0)),
            pl.BlockSpec((BN, 1), lambda i: (i, 0)),
            pl.BlockSpec((2, 1, d_in_h), lambda i: (0, 0, 0)),
            pl.BlockSpec((2 * d_in_h, d_out), lambda i: (0, 0)),
        ],
        out_specs=pl.BlockSpec((2, BN, doh), lambda i: (0, i, 0)),
        out_shape=jax.ShapeDtypeStruct((2, N, doh), jnp.float32),
    )(agg, ys, dis2, b2d, w)


def _tc_final(agg, ys, dis2, b2d, dh):
    def body(a_ref, ys_ref, dis_ref, b_ref, out_ref):
        dis = dis_ref[...]
        out_ref[0] = jnp.maximum((a_ref[0] + ys_ref[0]) * dis + b_ref[0], 0.0)
        out_ref[1] = jnp.maximum((a_ref[1] + ys_ref[1]) * dis + b_ref[1], 0.0)

    return pl.pallas_call(
        body,
        grid=(N // BN,),
        in_specs=[
            pl.BlockSpec((2, BN, dh), lambda i: (0, i, 0)),
            pl.BlockSpec((2, BN, dh), lambda i: (0, i, 0)),
            pl.BlockSpec((BN, 1), lambda i: (i, 0)),
            pl.BlockSpec((2, 1, dh), lambda i: (0, 0, 0)),
        ],
        out_specs=pl.BlockSpec((2, BN, dh), lambda i: (0, i, 0)),
        out_shape=jax.ShapeDtypeStruct((2, N, dh), jnp.float32),
    )(agg, ys, dis2, b2d)


def kernel(x, edge_index, edge_features, W1, b1, Wh, bh, W2, b2):
    src = edge_index[0].astype(jnp.int32)
    dst = edge_index[1].astype(jnp.int32)
    ew = edge_features.astype(jnp.float32)

    pad = E_PAD - E
    src_p = jnp.concatenate([src, jnp.zeros((pad,), jnp.int32)])
    dst_p = jnp.concatenate([dst, jnp.zeros((pad,), jnp.int32)])
    ew_p = jnp.concatenate([ew, jnp.zeros((pad,), jnp.float32)])
    src2 = jnp.stack([src_p, src_p + N]).reshape(2, ROWS_TOTAL, EDGE_COLS)
    dstr = dst_p.reshape(ROWS_TOTAL, EDGE_COLS)
    ewr = ew_p.reshape(ROWS_TOTAL, EDGE_COLS)

    ones16 = jnp.ones((2 * N, 16), jnp.float32)
    deg16 = _sc_agg(16)(ones16, src2, dstr, ewr)[:N]

    b1_2d = b1.reshape(2, 1, 128)
    bh_2d = bh.reshape(2, 1, 128)
    b2_2d = b2.reshape(2, 1, 64)

    ys1, dis2 = _tc_first(x, W1, deg16)
    a1 = _sc_agg(128)(ys1.reshape(2 * N, 128), src2, dstr, ewr)
    ys2 = _tc_mid(a1.reshape(2, N, 128), ys1, dis2, b1_2d, Wh, 128, 256)
    a2 = _sc_agg(128)(ys2.reshape(2 * N, 128), src2, dstr, ewr)
    ys3 = _tc_mid(a2.reshape(2, N, 128), ys2, dis2, bh_2d, W2, 128, 128)
    a3 = _sc_agg(64)(ys3.reshape(2 * N, 64), src2, dstr, ewr)
    out2 = _tc_final(a3.reshape(2, N, 64), ys3, dis2, b2_2d, 64)
    return jnp.transpose(out2, (1, 0, 2)).reshape(N, 128)


# speedup vs baseline: 4.7274x; 4.7274x over previous
import functools

import jax
import jax.numpy as jnp
from jax import lax
from jax.experimental import pallas as pl
from jax.experimental.pallas import tpu as pltpu
from jax.experimental.pallas import tpu_sc as plsc

N = 10000
E = 320000
LANES = 16
EDGE_COLS = 128
TILES = 16
ROWS_PER_TILE = 160
ROWS_TOTAL = TILES * ROWS_PER_TILE
E_PAD = ROWS_TOTAL * EDGE_COLS
NCHUNK = 624
ZCHUNK = 16
CR = 16

BN = 1000


@functools.cache
def _sc_agg(dh):
    assert dh % 128 == 0
    mesh = plsc.VectorSubcoreMesh(core_axis_name="c", subcore_axis_name="s")

    @functools.partial(
        pl.kernel,
        mesh=mesh,
        out_type=jax.ShapeDtypeStruct((2 * N, dh), jnp.float32),
        scratch_types=[
            pltpu.VMEM((CR, EDGE_COLS), jnp.int32),
            pltpu.VMEM((CR, EDGE_COLS), jnp.int32),
            pltpu.VMEM((CR, EDGE_COLS), jnp.float32),
            pltpu.VMEM((EDGE_COLS, dh), jnp.float32),
            pltpu.VMEM((ZCHUNK, dh), jnp.float32),
            pltpu.VMEM_SHARED((N, dh), jnp.float32),
            pltpu.SemaphoreType.DMA,
        ],
    )
    def agg(ys_hbm, src_hbm, dst_hbm, ew_hbm, out_hbm,
            src_v, dst_v, ew_v, rows_v, zero_v, acc_sh, sem):
        c = lax.axis_index("c")
        s = lax.axis_index("s")

        zf = jnp.zeros((LANES,), jnp.float32)

        def zrow(r, carry):
            for k in range(dh // LANES):
                zero_v[r, pl.ds(k * LANES, LANES)] = zf
            return carry

        lax.fori_loop(0, ZCHUNK, zrow, 0)
        n0 = s * NCHUNK
        nz = jnp.where(s == TILES - 1, (N - (TILES - 1) * NCHUNK) // ZCHUNK,
                       NCHUNK // ZCHUNK)

        def zcopy(t, carry):
            pltpu.sync_copy(zero_v, acc_sh.at[pl.ds(n0 + t * ZCHUNK, ZCHUNK)])
            return carry

        lax.fori_loop(0, nz, zcopy, 0)
        plsc.subcore_barrier()

        row0 = s * ROWS_PER_TILE

        def chunk(ci, carry):
            r0 = row0 + ci * CR
            pltpu.sync_copy(src_hbm.at[c, pl.ds(r0, CR)], src_v)
            pltpu.sync_copy(dst_hbm.at[pl.ds(r0, CR)], dst_v)
            pltpu.sync_copy(ew_hbm.at[pl.ds(r0, CR)], ew_v)

            def body(j, carry2):
                pltpu.async_copy(ys_hbm.at[src_v.at[j]], rows_v, sem).wait()

                def scale(g, inner):
                    wv = ew_v[j, pl.ds(g * LANES, LANES)]
                    e0 = g * LANES
                    for i in range(LANES):
                        w = wv[i]
                        for k in range(dh // LANES):
                            rows_v[e0 + i, pl.ds(k * LANES, LANES)] = (
                                rows_v[e0 + i, pl.ds(k * LANES, LANES)] * w)
                    return inner

                lax.fori_loop(0, EDGE_COLS // LANES, scale, 0)

                pltpu.sync_copy(rows_v, acc_sh.at[dst_v.at[j]], add=True)
                return carry2

            lax.fori_loop(0, CR, body, 0)
            return carry

        lax.fori_loop(0, ROWS_PER_TILE // CR, chunk, 0)
        plsc.subcore_barrier()

        last = N - (TILES - 1) * NCHUNK

        @pl.when(s < TILES - 1)
        def _():
            pltpu.sync_copy(acc_sh.at[pl.ds(n0, NCHUNK)],
                            out_hbm.at[pl.ds(c * N + n0, NCHUNK)])

        @pl.when(s == TILES - 1)
        def _():
            pltpu.sync_copy(acc_sh.at[pl.ds(n0, last)],
                            out_hbm.at[pl.ds(c * N + n0, last)])

    return agg


RPT_FULL = ROWS_TOTAL // 32


@functools.cache
def _sc_agg_full():
    dh = 128
    mesh = plsc.VectorSubcoreMesh(core_axis_name="c", subcore_axis_name="s")

    @functools.partial(
        pl.kernel,
        mesh=mesh,
        out_type=jax.ShapeDtypeStruct((2 * N, dh), jnp.float32),
        scratch_types=[
            pltpu.VMEM((CR, EDGE_COLS), jnp.int32),
            pltpu.VMEM((CR, EDGE_COLS), jnp.int32),
            pltpu.VMEM((CR, EDGE_COLS), jnp.float32),
            pltpu.VMEM((EDGE_COLS, dh), jnp.float32),
            pltpu.VMEM((ZCHUNK, dh), jnp.float32),
            pltpu.VMEM_SHARED((N, dh), jnp.float32),
            pltpu.SemaphoreType.DMA,
        ],
    )
    def agg(ys_hbm, src_hbm, dst_hbm, ew_hbm, out_hbm,
            src_v, dst_v, ew_v, rows_v, zero_v, acc_sh, sem):
        c = lax.axis_index("c")
        s = lax.axis_index("s")

        zf = jnp.zeros((LANES,), jnp.float32)

        def zrow(r, carry):
            for k in range(dh // LANES):
                zero_v[r, pl.ds(k * LANES, LANES)] = zf
            return carry

        lax.fori_loop(0, ZCHUNK, zrow, 0)
        n0 = s * NCHUNK
        nz = jnp.where(s == TILES - 1, (N - (TILES - 1) * NCHUNK) // ZCHUNK,
                       NCHUNK // ZCHUNK)

        def zcopy(t, carry):
            pltpu.sync_copy(zero_v, acc_sh.at[pl.ds(n0 + t * ZCHUNK, ZCHUNK)])
            return carry

        lax.fori_loop(0, nz, zcopy, 0)
        plsc.subcore_barrier()

        row0 = (c * TILES + s) * RPT_FULL

        def chunk(ci, carry):
            r0 = row0 + ci * CR
            pltpu.sync_copy(src_hbm.at[pl.ds(r0, CR)], src_v)
            pltpu.sync_copy(dst_hbm.at[pl.ds(r0, CR)], dst_v)
            pltpu.sync_copy(ew_hbm.at[pl.ds(r0, CR)], ew_v)

            def body(j, carry2):
                pltpu.async_copy(ys_hbm.at[src_v.at[j]], rows_v, sem).wait()

                def scale(g, inner):
                    wv = ew_v[j, pl.ds(g * LANES, LANES)]
                    e0 = g * LANES
                    for i in range(LANES):
                        w = wv[i]
                        for k in range(dh // LANES):
                            rows_v[e0 + i, pl.ds(k * LANES, LANES)] = (
                                rows_v[e0 + i, pl.ds(k * LANES, LANES)] * w)
                    return inner

                lax.fori_loop(0, EDGE_COLS // LANES, scale, 0)
                pltpu.sync_copy(rows_v, acc_sh.at[dst_v.at[j]], add=True)
                return carry2

            lax.fori_loop(0, CR, body, 0)
            return carry

        lax.fori_loop(0, RPT_FULL // CR, chunk, 0)
        plsc.subcore_barrier()

        last = N - (TILES - 1) * NCHUNK

        @pl.when(s < TILES - 1)
        def _():
            pltpu.sync_copy(acc_sh.at[pl.ds(n0, NCHUNK)],
                            out_hbm.at[pl.ds(c * N + n0, NCHUNK)])

        @pl.when(s == TILES - 1)
        def _():
            pltpu.sync_copy(acc_sh.at[pl.ds(n0, last)],
                            out_hbm.at[pl.ds(c * N + n0, last)])

    return agg


def _tc_first(x, w1, degp):
    def body(x_ref, w_ref, deg_ref, ys_ref, dis_ref):
        deg = deg_ref[0, :, 0:1] + deg_ref[1, :, 0:1] + 1.0
        dis = lax.rsqrt(deg)
        xw = jnp.dot(x_ref[...], w_ref[...],
                     preferred_element_type=jnp.float32)
        ys = xw * dis
        ys_ref[0] = ys[:, :128]
        ys_ref[1] = ys[:, 128:]
        dis_ref[...] = dis

    return pl.pallas_call(
        body,
        grid=(N // BN,),
        in_specs=[
            pl.BlockSpec((BN, 128), lambda i: (i, 0)),
            pl.BlockSpec((128, 256), lambda i: (0, 0)),
            pl.BlockSpec((2, BN, 128), lambda i: (0, i, 0)),
        ],
        out_specs=[
            pl.BlockSpec((2, BN, 128), lambda i: (0, i, 0)),
            pl.BlockSpec((BN, 1), lambda i: (i, 0)),
        ],
        out_shape=[
            jax.ShapeDtypeStruct((2, N, 128), jnp.float32),
            jax.ShapeDtypeStruct((N, 1), jnp.float32),
        ],
    )(x, w1, degp)


def _tc_mid(agg, ys, dis2, b2d, w, d_in_h, d_out, split_out):
    doh = d_out // 2

    def body(a_ref, ys_ref, dis_ref, b_ref, w_ref, out_ref):
        dis = dis_ref[...]
        h0 = jnp.maximum((a_ref[0] + ys_ref[0]) * dis + b_ref[0], 0.0)
        h1 = jnp.maximum((a_ref[1] + ys_ref[1]) * dis + b_ref[1], 0.0)
        out = jnp.dot(h0, w_ref[:d_in_h, :],
                      preferred_element_type=jnp.float32)
        out = out + jnp.dot(h1, w_ref[d_in_h:, :],
                            preferred_element_type=jnp.float32)
        ysn = out * dis
        if split_out:
            out_ref[0] = ysn[:, :doh]
            out_ref[1] = ysn[:, doh:]
        else:
            out_ref[...] = ysn

    if split_out:
        out_spec = pl.BlockSpec((2, BN, doh), lambda i: (0, i, 0))
        out_shape = jax.ShapeDtypeStruct((2, N, doh), jnp.float32)
    else:
        out_spec = pl.BlockSpec((BN, d_out), lambda i: (i, 0))
        out_shape = jax.ShapeDtypeStruct((N, d_out), jnp.float32)

    return pl.pallas_call(
        body,
        grid=(N // BN,),
        in_specs=[
            pl.BlockSpec((2, BN, d_in_h), lambda i: (0, i, 0)),
            pl.BlockSpec((2, BN, d_in_h), lambda i: (0, i, 0)),
            pl.BlockSpec((BN, 1), lambda i: (i, 0)),
            pl.BlockSpec((2, 1, d_in_h), lambda i: (0, 0, 0)),
            pl.BlockSpec((2 * d_in_h, d_out), lambda i: (0, 0)),
        ],
        out_specs=out_spec,
        out_shape=out_shape,
    )(agg, ys, dis2, b2d, w)


def _tc_final(aggp, ys, dis2, b2d):
    def body(a_ref, ys_ref, dis_ref, b_ref, out_ref):
        dis = dis_ref[...]
        a = a_ref[0] + a_ref[1]
        out_ref[...] = jnp.maximum((a + ys_ref[...]) * dis + b_ref[...], 0.0)

    return pl.pallas_call(
        body,
        grid=(N // BN,),
        in_specs=[
            pl.BlockSpec((2, BN, 128), lambda i: (0, i, 0)),
            pl.BlockSpec((BN, 128), lambda i: (i, 0)),
            pl.BlockSpec((BN, 1), lambda i: (i, 0)),
            pl.BlockSpec((1, 128), lambda i: (0, 0)),
        ],
        out_specs=pl.BlockSpec((BN, 128), lambda i: (i, 0)),
        out_shape=jax.ShapeDtypeStruct((N, 128), jnp.float32),
    )(aggp, ys, dis2, b2d)


def kernel(x, edge_index, edge_features, W1, b1, Wh, bh, W2, b2):
    src = edge_index[0].astype(jnp.int32)
    dst = edge_index[1].astype(jnp.int32)
    ew = edge_features.astype(jnp.float32)

    pad = E_PAD - E
    src_p = jnp.concatenate([src, jnp.zeros((pad,), jnp.int32)])
    dst_p = jnp.concatenate([dst, jnp.zeros((pad,), jnp.int32)])
    ew_p = jnp.concatenate([ew, jnp.zeros((pad,), jnp.float32)])
    src2 = jnp.stack([src_p, src_p + N]).reshape(2, ROWS_TOTAL, EDGE_COLS)
    dstr = dst_p.reshape(ROWS_TOTAL, EDGE_COLS)
    ewr = ew_p.reshape(ROWS_TOTAL, EDGE_COLS)

    ones128 = jnp.ones((N, 128), jnp.float32)
    degp = _sc_agg_full()(ones128, src2[0], dstr, ewr).reshape(2, N, 128)

    b1_2d = b1.reshape(2, 1, 128)
    bh_2d = bh.reshape(2, 1, 128)
    b2_2d = b2.reshape(1, 128)

    ys1, dis2 = _tc_first(x, W1, degp)
    a1 = _sc_agg(128)(ys1.reshape(2 * N, 128), src2, dstr, ewr)
    ys2 = _tc_mid(a1.reshape(2, N, 128), ys1, dis2, b1_2d, Wh, 128, 256,
                  split_out=True)
    a2 = _sc_agg(128)(ys2.reshape(2 * N, 128), src2, dstr, ewr)
    ys3 = _tc_mid(a2.reshape(2, N, 128), ys2, dis2, bh_2d, W2, 128, 128,
                  split_out=False)
    a3p = _sc_agg_full()(ys3, src2[0], dstr, ewr).reshape(2, N, 128)
    return _tc_final(a3p, ys3, dis2, b2_2d)


# speedup vs baseline: 5.8962x; 1.2472x over previous
import functools

import jax
import jax.numpy as jnp
from jax import lax
from jax.experimental import pallas as pl
from jax.experimental.pallas import tpu as pltpu
from jax.experimental.pallas import tpu_sc as plsc

N = 10000
E = 320000
LANES = 16
EDGE_COLS = 128
TILES = 16
ROWS_PER_TILE = 160
ROWS_TOTAL = TILES * ROWS_PER_TILE
E_PAD = ROWS_TOTAL * EDGE_COLS
RPT_FULL = ROWS_TOTAL // 32
NCHUNK = 624
ZCHUNK = 16
CR = 16
CRH = CR // 2
DH = 128

BN = 1000


@functools.cache
def _sc_pass(feature_split):
    mesh = plsc.VectorSubcoreMesh(core_axis_name="c", subcore_axis_name="s")
    rpt = ROWS_PER_TILE if feature_split else RPT_FULL

    @functools.partial(
        pl.kernel,
        mesh=mesh,
        out_type=jax.ShapeDtypeStruct((2 * N, DH), jnp.float32),
        scratch_types=[
            pltpu.VMEM((CR, EDGE_COLS), jnp.int32),
            pltpu.VMEM((CR, EDGE_COLS), jnp.int32),
            pltpu.VMEM((CR, EDGE_COLS), jnp.float32),
            pltpu.VMEM((2, EDGE_COLS, DH), jnp.float32),
            pltpu.VMEM((ZCHUNK, DH), jnp.float32),
            pltpu.VMEM_SHARED((N, DH), jnp.float32),
            pltpu.SemaphoreType.DMA,
            pltpu.SemaphoreType.DMA,
        ],
    )
    def agg(ys_hbm, src_hbm, dst_hbm, ew_hbm, out_hbm,
            src_v, dst_v, ew_v, rows_v, zero_v, acc_sh, sem0, sem1):
        c = lax.axis_index("c")
        s = lax.axis_index("s")

        zf = jnp.zeros((LANES,), jnp.float32)

        def zrow(r, carry):
            for k in range(DH // LANES):
                zero_v[r, pl.ds(k * LANES, LANES)] = zf
            return carry

        lax.fori_loop(0, ZCHUNK, zrow, 0)
        n0 = s * NCHUNK
        nz = jnp.where(s == TILES - 1, (N - (TILES - 1) * NCHUNK) // ZCHUNK,
                       NCHUNK // ZCHUNK)

        def zcopy(t, carry):
            pltpu.sync_copy(zero_v, acc_sh.at[pl.ds(n0 + t * ZCHUNK, ZCHUNK)])
            return carry

        lax.fori_loop(0, nz, zcopy, 0)
        plsc.subcore_barrier()

        if feature_split:
            row0 = s * ROWS_PER_TILE
        else:
            row0 = (c * TILES + s) * RPT_FULL

        def gather_start(b, j, sem):
            pltpu.async_copy(ys_hbm.at[src_v.at[j]], rows_v.at[b], sem)

        def gather_wait(b, j, sem):
            pltpu.make_async_copy(ys_hbm.at[src_v.at[j]], rows_v.at[b],
                                  sem).wait()

        def process(b, j):
            def scale(g, inner):
                wv = ew_v[j, pl.ds(g * LANES, LANES)]
                e0 = g * LANES
                for i in range(LANES):
                    w = wv[i]
                    for k in range(DH // LANES):
                        rows_v[b, e0 + i, pl.ds(k * LANES, LANES)] = (
                            rows_v[b, e0 + i, pl.ds(k * LANES, LANES)] * w)
                return inner

            lax.fori_loop(0, EDGE_COLS // LANES, scale, 0)
            pltpu.sync_copy(rows_v.at[b], acc_sh.at[dst_v.at[j]], add=True)

        def chunk(ci, carry):
            r0 = row0 + ci * CR
            if feature_split:
                pltpu.sync_copy(src_hbm.at[c, pl.ds(r0, CR)], src_v)
            else:
                pltpu.sync_copy(src_hbm.at[pl.ds(r0, CR)], src_v)
            pltpu.sync_copy(dst_hbm.at[pl.ds(r0, CR)], dst_v)
            pltpu.sync_copy(ew_hbm.at[pl.ds(r0, CR)], ew_v)

            gather_start(0, 0, sem0)

            def pair(p, c2):
                j0 = p * 2
                gather_start(1, j0 + 1, sem1)
                gather_wait(0, j0, sem0)
                process(0, j0)

                @pl.when(p < CRH - 1)
                def _():
                    gather_start(0, j0 + 2, sem0)

                gather_wait(1, j0 + 1, sem1)
                process(1, j0 + 1)
                return c2

            lax.fori_loop(0, CRH, pair, 0)
            return carry

        lax.fori_loop(0, rpt // CR, chunk, 0)
        plsc.subcore_barrier()

        last = N - (TILES - 1) * NCHUNK

        @pl.when(s < TILES - 1)
        def _():
            pltpu.sync_copy(acc_sh.at[pl.ds(n0, NCHUNK)],
                            out_hbm.at[pl.ds(c * N + n0, NCHUNK)])

        @pl.when(s == TILES - 1)
        def _():
            pltpu.sync_copy(acc_sh.at[pl.ds(n0, last)],
                            out_hbm.at[pl.ds(c * N + n0, last)])

    return agg


def _tc_first(x, w1, degp):
    def body(x_ref, w_ref, deg_ref, ys_ref, dis_ref):
        deg = deg_ref[0, :, 0:1] + deg_ref[1, :, 0:1] + 1.0
        dis = lax.rsqrt(deg)
        xw = jnp.dot(x_ref[...], w_ref[...],
                     preferred_element_type=jnp.float32)
        ys = xw * dis
        ys_ref[0] = ys[:, :128]
        ys_ref[1] = ys[:, 128:]
        dis_ref[...] = dis

    return pl.pallas_call(
        body,
        grid=(N // BN,),
        in_specs=[
            pl.BlockSpec((BN, 128), lambda i: (i, 0)),
            pl.BlockSpec((128, 256), lambda i: (0, 0)),
            pl.BlockSpec((2, BN, 128), lambda i: (0, i, 0)),
        ],
        out_specs=[
            pl.BlockSpec((2, BN, 128), lambda i: (0, i, 0)),
            pl.BlockSpec((BN, 1), lambda i: (i, 0)),
        ],
        out_shape=[
            jax.ShapeDtypeStruct((2, N, 128), jnp.float32),
            jax.ShapeDtypeStruct((N, 1), jnp.float32),
        ],
    )(x, w1, degp)


def _tc_mid(agg, ys, dis2, b2d, w, d_in_h, d_out, split_out):
    doh = d_out // 2

    def body(a_ref, ys_ref, dis_ref, b_ref, w_ref, out_ref):
        dis = dis_ref[...]
        h0 = jnp.maximum((a_ref[0] + ys_ref[0]) * dis + b_ref[0], 0.0)
        h1 = jnp.maximum((a_ref[1] + ys_ref[1]) * dis + b_ref[1], 0.0)
        out = jnp.dot(h0, w_ref[:d_in_h, :],
                      preferred_element_type=jnp.float32)
        out = out + jnp.dot(h1, w_ref[d_in_h:, :],
                            preferred_element_type=jnp.float32)
        ysn = out * dis
        if split_out:
            out_ref[0] = ysn[:, :doh]
            out_ref[1] = ysn[:, doh:]
        else:
            out_ref[...] = ysn

    if split_out:
        out_spec = pl.BlockSpec((2, BN, doh), lambda i: (0, i, 0))
        out_shape = jax.ShapeDtypeStruct((2, N, doh), jnp.float32)
    else:
        out_spec = pl.BlockSpec((BN, d_out), lambda i: (i, 0))
        out_shape = jax.ShapeDtypeStruct((N, d_out), jnp.float32)

    return pl.pallas_call(
        body,
        grid=(N // BN,),
        in_specs=[
            pl.BlockSpec((2, BN, d_in_h), lambda i: (0, i, 0)),
            pl.BlockSpec((2, BN, d_in_h), lambda i: (0, i, 0)),
            pl.BlockSpec((BN, 1), lambda i: (i, 0)),
            pl.BlockSpec((2, 1, d_in_h), lambda i: (0, 0, 0)),
            pl.BlockSpec((2 * d_in_h, d_out), lambda i: (0, 0)),
        ],
        out_specs=out_spec,
        out_shape=out_shape,
    )(agg, ys, dis2, b2d, w)


def _tc_final(aggp, ys, dis2, b2d):
    def body(a_ref, ys_ref, dis_ref, b_ref, out_ref):
        dis = dis_ref[...]
        a = a_ref[0] + a_ref[1]
        out_ref[...] = jnp.maximum((a + ys_ref[...]) * dis + b_ref[...], 0.0)

    return pl.pallas_call(
        body,
        grid=(N // BN,),
        in_specs=[
            pl.BlockSpec((2, BN, 128), lambda i: (0, i, 0)),
            pl.BlockSpec((BN, 128), lambda i: (i, 0)),
            pl.BlockSpec((BN, 1), lambda i: (i, 0)),
            pl.BlockSpec((1, 128), lambda i: (0, 0)),
        ],
        out_specs=pl.BlockSpec((BN, 128), lambda i: (i, 0)),
        out_shape=jax.ShapeDtypeStruct((N, 128), jnp.float32),
    )(aggp, ys, dis2, b2d)


def kernel(x, edge_index, edge_features, W1, b1, Wh, bh, W2, b2):
    src = edge_index[0].astype(jnp.int32)
    dst = edge_index[1].astype(jnp.int32)
    ew = edge_features.astype(jnp.float32)

    pad = E_PAD - E
    src_p = jnp.concatenate([src, jnp.zeros((pad,), jnp.int32)])
    dst_p = jnp.concatenate([dst, jnp.zeros((pad,), jnp.int32)])
    ew_p = jnp.concatenate([ew, jnp.zeros((pad,), jnp.float32)])
    src2 = jnp.stack([src_p, src_p + N]).reshape(2, ROWS_TOTAL, EDGE_COLS)
    dstr = dst_p.reshape(ROWS_TOTAL, EDGE_COLS)
    ewr = ew_p.reshape(ROWS_TOTAL, EDGE_COLS)

    ones128 = jnp.ones((N, 128), jnp.float32)
    degp = _sc_pass(False)(ones128, src2[0], dstr, ewr).reshape(2, N, 128)

    b1_2d = b1.reshape(2, 1, 128)
    bh_2d = bh.reshape(2, 1, 128)
    b2_2d = b2.reshape(1, 128)

    ys1, dis2 = _tc_first(x, W1, degp)
    a1 = _sc_pass(True)(ys1.reshape(2 * N, 128), src2, dstr, ewr)
    ys2 = _tc_mid(a1.reshape(2, N, 128), ys1, dis2, b1_2d, Wh, 128, 256,
                  split_out=True)
    a2 = _sc_pass(True)(ys2.reshape(2 * N, 128), src2, dstr, ewr)
    ys3 = _tc_mid(a2.reshape(2, N, 128), ys2, dis2, bh_2d, W2, 128, 128,
                  split_out=False)
    a3p = _sc_pass(False)(ys3, src2[0], dstr, ewr).reshape(2, N, 128)
    return _tc_final(a3p, ys3, dis2, b2_2d)
